# baseline (speedup 1.0000x reference)
import numpy as np
import jax
import jax.numpy as jnp
from jax import lax
from jax.experimental import pallas as pl
from jax.experimental.pallas import tpu as pltpu
from jax.experimental.pallas import tpu_sc as plsc

N_BINS = 10
NC = 2
NS = 16
L = 16
NW = NC * NS
ROWS = 32 * 512
COLS = 512
ROWS_SC = 8192
ROWS_TC = ROWS - ROWS_SC
ROWS_W = ROWS_SC // NW
CROWS = 32
NCHUNK = ROWS_W // CROWS
VPC = CROWS * COLS // L
TC_BR = 512
TC_G = ROWS_TC // TC_BR

_START = -1e-6
_INV_STEP = np.float32(1.0 / ((1.0 - _START) / 10.0))
_C0 = np.float32(-_START * (1.0 / ((1.0 - _START) / 10.0)))
_MAGIC = np.float32(12582912.0)


def _body(o_hbm, l_hbm, he_hbm, out_hbm,
          obuf0, obuf1, lbuf0, lbuf1, htab, ptab, ttab, pbuf,
          so0, so1, sl0, sl1):
    wid = lax.axis_index("s") * NC + lax.axis_index("c")
    base = wid * ROWS_W
    obufs = (obuf0, obuf1)
    lbufs = (lbuf0, lbuf1)
    sos = (so0, so1)
    sls = (sl0, sl1)

    pltpu.sync_copy(he_hbm, htab)

    zf = jnp.zeros((L,), jnp.float32)
    zi = jnp.zeros((L,), jnp.int32)
    for ln in range(NS):
        ptab[pl.ds(ln * L, L)] = zf
        ttab[pl.ds(ln * L, L)] = zi

    lane_base = lax.iota(jnp.int32, L) * L
    lane_m1 = lane_base - 1

    def start(s, g):
        r0 = base + g * CROWS
        pltpu.async_copy(o_hbm.at[pl.ds(r0, CROWS)], obufs[s], sos[s])
        pltpu.async_copy(l_hbm.at[pl.ds(r0, CROWS)], lbufs[s], sls[s])

    def wait(s):
        pltpu.make_async_copy(o_hbm.at[pl.ds(0, CROWS)], obufs[s], sos[s]).wait()
        pltpu.make_async_copy(l_hbm.at[pl.ds(0, CROWS)], lbufs[s], sls[s]).wait()

    htab_vec = htab[pl.ds(0, L)]
    gdn = lax.GatherDimensionNumbers(
        offset_dims=(), collapsed_slice_dims=(0,), start_index_map=(0,))

    def compute(s):
        obuf = obufs[s]
        lbuf = lbufs[s]

        @plsc.parallel_loop(0, VPC, unroll=8)
        def _(i):
            r = lax.shift_right_logical(i, 5)
            c = jnp.bitwise_and(i, 31) * L
            x = obuf[r, pl.ds(c, L)]
            lb = lbuf[r, pl.ds(c, L)]
            y = (x * _INV_STEP + _C0) + _MAGIC
            cand = jnp.bitwise_and(plsc.bitcast(y, jnp.int32), 15)
            hi = lax.gather(htab_vec, cand[:, None], dimension_numbers=gdn,
                            slice_sizes=(1,),
                            mode=lax.GatherScatterMode.PROMISE_IN_BOUNDS)
            idx = cand + jnp.where(x <= hi, lane_m1, lane_base)
            plsc.addupdate_scatter(ptab, [idx], x)
            tc = jnp.where(lb > 0.5, jnp.int32(65537), jnp.int32(1))
            plsc.addupdate_scatter(ttab, [idx], tc)

    start(0, 0)
    start(1, 1)

    def outer(h, _):
        for s in range(2):
            g = h * 2 + s
            wait(s)
            compute(s)

            @pl.when(g + 2 < NCHUNK)
            def _():
                start(s, g + 2)
        return 0

    lax.fori_loop(0, NCHUNK // 2, outer, 0)

    pacc = jnp.zeros((L,), jnp.float32)
    cacc = jnp.zeros((L,), jnp.int32)
    tacc = jnp.zeros((L,), jnp.int32)
    for ln in range(NS):
        pacc = pacc + ptab[pl.ds(ln * L, L)]
        tv = ttab[pl.ds(ln * L, L)]
        cacc = cacc + jnp.bitwise_and(tv, 0xFFFF)
        tacc = tacc + lax.shift_right_logical(tv, 16)
    pbuf[pl.ds(0, L)] = pacc
    pbuf[pl.ds(L, L)] = tacc.astype(jnp.float32)
    pbuf[pl.ds(2 * L, L)] = cacc.astype(jnp.float32)
    pltpu.sync_copy(pbuf, out_hbm.at[wid])


def _tc_body(hb_ref, x_ref, lb_ref, out_ref):
    x = x_ref[...]
    lb = lb_ref[...]
    lm = lb > 0.5
    for j in range(N_BINS - 1):
        hj = hb_ref[j + 1]
        m = x > hj
        out_ref[0, j, :] = jnp.sum(jnp.where(m, x, 0.0), axis=0)
        out_ref[0, 9 + j, :] = jnp.sum(jnp.where(m, 1.0, 0.0), axis=0)
        out_ref[0, 18 + j, :] = jnp.sum(
            jnp.where(jnp.logical_and(m, lm), 1.0, 0.0), axis=0)
    out_ref[0, 27, :] = jnp.sum(x, axis=0)
    out_ref[0, 28, :] = jnp.sum(jnp.where(lm, 1.0, 0.0), axis=0)
    out_ref[0, 29, :] = jnp.zeros((COLS,), jnp.float32)
    out_ref[0, 30, :] = jnp.zeros((COLS,), jnp.float32)
    out_ref[0, 31, :] = jnp.zeros((COLS,), jnp.float32)


def kernel(outputs, labels):
    o2 = outputs.reshape(ROWS, COLS)
    l2 = labels.reshape(ROWS, COLS)
    linspace = jnp.linspace(-1e-06, 1.0, N_BINS + 1)
    high_ext = jnp.concatenate([
        jnp.array([-jnp.inf], jnp.float32),
        linspace[1:].astype(jnp.float32),
        jnp.full((L - N_BINS - 1,), jnp.inf, jnp.float32),
    ])

    k = pl.kernel(
        _body,
        out_type=jax.ShapeDtypeStruct((NW, 3 * L), jnp.float32),
        mesh=plsc.VectorSubcoreMesh(core_axis_name="c", subcore_axis_name="s"),
        compiler_params=pltpu.CompilerParams(needs_layout_passes=False),
        scratch_types=[
            pltpu.VMEM((CROWS, COLS), jnp.float32),
            pltpu.VMEM((CROWS, COLS), jnp.float32),
            pltpu.VMEM((CROWS, COLS), jnp.float32),
            pltpu.VMEM((CROWS, COLS), jnp.float32),
            pltpu.VMEM((L,), jnp.float32),
            pltpu.VMEM((NS * L,), jnp.float32),
            pltpu.VMEM((NS * L,), jnp.int32),
            pltpu.VMEM((3 * L,), jnp.float32),
            pltpu.SemaphoreType.DMA,
            pltpu.SemaphoreType.DMA,
            pltpu.SemaphoreType.DMA,
            pltpu.SemaphoreType.DMA,
        ],
    )
    k_tc = pl.pallas_call(
        _tc_body,
        grid=(TC_G,),
        in_specs=[
            pl.BlockSpec(memory_space=pltpu.SMEM),
            pl.BlockSpec((TC_BR, COLS), lambda g: (ROWS_SC // TC_BR + g, 0)),
            pl.BlockSpec((TC_BR, COLS), lambda g: (ROWS_SC // TC_BR + g, 0)),
        ],
        out_specs=pl.BlockSpec((1, 32, COLS), lambda g: (g, 0, 0)),
        out_shape=jax.ShapeDtypeStruct((TC_G, 32, COLS), jnp.float32),
    )

    parts = k(o2, l2, high_ext)
    tc_out = k_tc(high_ext, o2, l2)

    s = parts.sum(axis=0)
    sc_prob = s[0:N_BINS]
    sc_tp = s[L:L + N_BINS]
    sc_cnt = s[2 * L:2 * L + N_BINS]

    t = tc_out.sum(axis=(0, 2))
    P, C, TP = t[0:9], t[9:18], t[18:27]
    TX, TL = t[27], t[28]
    n_tc = jnp.float32(ROWS_TC * COLS)
    tc_prob = jnp.concatenate([(TX - P[0])[None], P[0:8] - P[1:9], P[8:9]])
    tc_cnt = jnp.concatenate([(n_tc - C[0])[None], C[0:8] - C[1:9], C[8:9]])
    tc_tp = jnp.concatenate([(TL - TP[0])[None], TP[0:8] - TP[1:9], TP[8:9]])

    return sc_prob + tc_prob, sc_tp + tc_tp, sc_cnt + tc_cnt


# speedup vs baseline: 1.2276x; 1.2276x over previous
import numpy as np
import jax
import jax.numpy as jnp
from jax import lax
from jax.experimental import pallas as pl
from jax.experimental.pallas import tpu as pltpu
from jax.experimental.pallas import tpu_sc as plsc

N_BINS = 10
NC = 2
NS = 16
L = 16
NW = NC * NS
ROWS = 32 * 512
COLS = 512
ROWS_SC = 10240
ROWS_TC = ROWS - ROWS_SC
ROWS_W = ROWS_SC // NW
CROWS = 32
NCHUNK = ROWS_W // CROWS
VPC = CROWS * COLS // L
TC_BR = 512
TC_G = ROWS_TC // TC_BR

_START = -1e-6
_INV_STEP = np.float32(1.0 / ((1.0 - _START) / 10.0))
_C0 = np.float32(-_START * (1.0 / ((1.0 - _START) / 10.0)))
_MAGIC = np.float32(12582912.0)


def _body(o_hbm, l_hbm, he_hbm, out_hbm,
          obuf0, obuf1, lbuf0, lbuf1, htab, ptab, ttab, pbuf,
          so0, so1, sl0, sl1):
    wid = lax.axis_index("s") * NC + lax.axis_index("c")
    base = wid * ROWS_W
    obufs = (obuf0, obuf1)
    lbufs = (lbuf0, lbuf1)
    sos = (so0, so1)
    sls = (sl0, sl1)

    pltpu.sync_copy(he_hbm, htab)

    zf = jnp.zeros((L,), jnp.float32)
    zi = jnp.zeros((L,), jnp.int32)
    for ln in range(NS):
        ptab[pl.ds(ln * L, L)] = zf
        ttab[pl.ds(ln * L, L)] = zi

    lane_base = lax.iota(jnp.int32, L) * L
    lane_m1 = lane_base - 1

    def start(s, g):
        r0 = base + g * CROWS
        pltpu.async_copy(o_hbm.at[pl.ds(r0, CROWS)], obufs[s], sos[s])
        pltpu.async_copy(l_hbm.at[pl.ds(r0, CROWS)], lbufs[s], sls[s])

    def wait(s):
        pltpu.make_async_copy(o_hbm.at[pl.ds(0, CROWS)], obufs[s], sos[s]).wait()
        pltpu.make_async_copy(l_hbm.at[pl.ds(0, CROWS)], lbufs[s], sls[s]).wait()

    htab_vec = htab[pl.ds(0, L)]
    gdn = lax.GatherDimensionNumbers(
        offset_dims=(), collapsed_slice_dims=(0,), start_index_map=(0,))

    def compute(s):
        obuf = obufs[s]
        lbuf = lbufs[s]

        @plsc.parallel_loop(0, VPC, unroll=8)
        def _(i):
            r = lax.shift_right_logical(i, 5)
            c = jnp.bitwise_and(i, 31) * L
            x = obuf[r, pl.ds(c, L)]
            lb = lbuf[r, pl.ds(c, L)]
            y = (x * _INV_STEP + _C0) + _MAGIC
            cand = jnp.bitwise_and(plsc.bitcast(y, jnp.int32), 15)
            hi = lax.gather(htab_vec, cand[:, None], dimension_numbers=gdn,
                            slice_sizes=(1,),
                            mode=lax.GatherScatterMode.PROMISE_IN_BOUNDS)
            idx = cand + jnp.where(x <= hi, lane_m1, lane_base)
            plsc.addupdate_scatter(ptab, [idx], x)
            tc = jnp.where(lb > 0.5, jnp.int32(65537), jnp.int32(1))
            plsc.addupdate_scatter(ttab, [idx], tc)

    start(0, 0)
    start(1, 1)

    def outer(h, _):
        for s in range(2):
            g = h * 2 + s
            wait(s)
            compute(s)

            @pl.when(g + 2 < NCHUNK)
            def _():
                start(s, g + 2)
        return 0

    lax.fori_loop(0, NCHUNK // 2, outer, 0)

    pacc = jnp.zeros((L,), jnp.float32)
    cacc = jnp.zeros((L,), jnp.int32)
    tacc = jnp.zeros((L,), jnp.int32)
    for ln in range(NS):
        pacc = pacc + ptab[pl.ds(ln * L, L)]
        tv = ttab[pl.ds(ln * L, L)]
        cacc = cacc + jnp.bitwise_and(tv, 0xFFFF)
        tacc = tacc + lax.shift_right_logical(tv, 16)
    pbuf[pl.ds(0, L)] = pacc
    pbuf[pl.ds(L, L)] = tacc.astype(jnp.float32)
    pbuf[pl.ds(2 * L, L)] = cacc.astype(jnp.float32)
    pltpu.sync_copy(pbuf, out_hbm.at[wid])


def _tc_body(hb_ref, x_ref, lb_ref, out_ref):
    x = x_ref[...]
    lb = lb_ref[...]
    lmf = (lb > 0.5).astype(jnp.float32)
    for j in range(N_BINS - 1):
        hj = hb_ref[j + 1]
        mf = (x > hj).astype(jnp.float32)
        out_ref[0, j, :] = jnp.sum(x * mf, axis=0)
        out_ref[0, 9 + j, :] = jnp.sum(mf, axis=0)
        out_ref[0, 18 + j, :] = jnp.sum(lmf * mf, axis=0)
    out_ref[0, 27, :] = jnp.sum(x, axis=0)
    out_ref[0, 28, :] = jnp.sum(lmf, axis=0)
    out_ref[0, 29, :] = jnp.zeros((COLS,), jnp.float32)
    out_ref[0, 30, :] = jnp.zeros((COLS,), jnp.float32)
    out_ref[0, 31, :] = jnp.zeros((COLS,), jnp.float32)


def kernel(outputs, labels):
    o2 = outputs.reshape(ROWS, COLS)
    l2 = labels.reshape(ROWS, COLS)
    linspace = jnp.linspace(-1e-06, 1.0, N_BINS + 1)
    high_ext = jnp.concatenate([
        jnp.array([-jnp.inf], jnp.float32),
        linspace[1:].astype(jnp.float32),
        jnp.full((L - N_BINS - 1,), jnp.inf, jnp.float32),
    ])

    k = pl.kernel(
        _body,
        out_type=jax.ShapeDtypeStruct((NW, 3 * L), jnp.float32),
        mesh=plsc.VectorSubcoreMesh(core_axis_name="c", subcore_axis_name="s"),
        compiler_params=pltpu.CompilerParams(needs_layout_passes=False),
        scratch_types=[
            pltpu.VMEM((CROWS, COLS), jnp.float32),
            pltpu.VMEM((CROWS, COLS), jnp.float32),
            pltpu.VMEM((CROWS, COLS), jnp.float32),
            pltpu.VMEM((CROWS, COLS), jnp.float32),
            pltpu.VMEM((L,), jnp.float32),
            pltpu.VMEM((NS * L,), jnp.float32),
            pltpu.VMEM((NS * L,), jnp.int32),
            pltpu.VMEM((3 * L,), jnp.float32),
            pltpu.SemaphoreType.DMA,
            pltpu.SemaphoreType.DMA,
            pltpu.SemaphoreType.DMA,
            pltpu.SemaphoreType.DMA,
        ],
    )
    k_tc = pl.pallas_call(
        _tc_body,
        grid=(TC_G,),
        in_specs=[
            pl.BlockSpec(memory_space=pltpu.SMEM),
            pl.BlockSpec((TC_BR, COLS), lambda g: (ROWS_SC // TC_BR + g, 0)),
            pl.BlockSpec((TC_BR, COLS), lambda g: (ROWS_SC // TC_BR + g, 0)),
        ],
        out_specs=pl.BlockSpec((1, 32, COLS), lambda g: (g, 0, 0)),
        out_shape=jax.ShapeDtypeStruct((TC_G, 32, COLS), jnp.float32),
    )

    parts = k(o2, l2, high_ext)
    tc_out = k_tc(high_ext, o2, l2)

    s = parts.sum(axis=0)
    sc_prob = s[0:N_BINS]
    sc_tp = s[L:L + N_BINS]
    sc_cnt = s[2 * L:2 * L + N_BINS]

    t = tc_out.sum(axis=(0, 2))
    P, C, TP = t[0:9], t[9:18], t[18:27]
    TX, TL = t[27], t[28]
    n_tc = jnp.float32(ROWS_TC * COLS)
    tc_prob = jnp.concatenate([(TX - P[0])[None], P[0:8] - P[1:9], P[8:9]])
    tc_cnt = jnp.concatenate([(n_tc - C[0])[None], C[0:8] - C[1:9], C[8:9]])
    tc_tp = jnp.concatenate([(TL - TP[0])[None], TP[0:8] - TP[1:9], TP[8:9]])

    return sc_prob + tc_prob, sc_tp + tc_tp, sc_cnt + tc_cnt
